# baseline (speedup 1.0000x reference)
import functools

import jax
import jax.numpy as jnp
from jax import lax
from jax.experimental import pallas as pl
from jax.experimental.pallas import tpu as pltpu
from jax.experimental.pallas import tpu_sc as plsc

NC = 2
NS = 16
L = 16
NW = NC * NS

CHUNK = 32
IB = 8
SROW = 128
H2 = 64

CCHUNK = 128
CIB = 8


def _erf(z):
  az = jnp.abs(z)
  t = 1.0 / (1.0 + 0.3275911 * az)
  poly = t * (0.254829592 + t * (-0.284496736 + t * (1.421413741
              + t * (-1.453152027 + t * 1.061405429))))
  e = poly * jnp.exp(-az * az)
  pos = 1.0 - e
  return jnp.where(z < 0.0, -pos, pos)


def _gelu(x):
  return 0.5 * x * (1.0 + _erf(x * 0.7071067811865476))


def _gelu_fast(x):
  return x / (1.0 + jnp.exp(-1.702 * x))


def _pre_body(h_ref, w1_ref, q_ref, b1_ref, a_ref, b_ref):
  h = h_ref[...]
  w1a = w1_ref[0:128, :]
  w1b = w1_ref[128:256, :]
  w1c = w1_ref[256:384, :]
  qc = jnp.dot(q_ref[...], w1c, preferred_element_type=jnp.float32) + b1_ref[...]
  a_ref[...] = jnp.dot(h, w1a, preferred_element_type=jnp.float32) + qc
  b_ref[...] = jnp.dot(h, w1b, preferred_element_type=jnp.float32)


def _pre(h, w1, q, b1, n, hdim):
  blk = 1000
  grid = (n // blk,)
  return pl.pallas_call(
      _pre_body,
      grid=grid,
      in_specs=[
          pl.BlockSpec((blk, hdim), lambda i: (i, 0)),
          pl.BlockSpec((3 * hdim, hdim), lambda i: (0, 0)),
          pl.BlockSpec((1, hdim), lambda i: (0, 0)),
          pl.BlockSpec((1, hdim), lambda i: (0, 0)),
      ],
      out_specs=[
          pl.BlockSpec((blk, hdim), lambda i: (i, 0)),
          pl.BlockSpec((blk, hdim), lambda i: (i, 0)),
      ],
      out_shape=[
          jax.ShapeDtypeStruct((n, hdim), jnp.float32),
          jax.ShapeDtypeStruct((n, hdim), jnp.float32),
      ],
  )(h, w1, q, b1)


def _sc_counts(n_pad, chunks_per_tile):
  mesh = plsc.VectorSubcoreMesh(core_axis_name="c", subcore_axis_name="s",
                                num_cores=NC, num_subcores=NS)
  blocks = chunks_per_tile // CIB

  @functools.partial(
      pl.kernel,
      out_type=jax.ShapeDtypeStruct((NC, NS, n_pad), jnp.float32),
      mesh=mesh,
      compiler_params=pltpu.CompilerParams(needs_layout_passes=False),
      scratch_types=[
          pltpu.VMEM((CIB, CCHUNK), jnp.int32),
          pltpu.VMEM((n_pad,), jnp.float32),
      ],
  )
  def sc_counts(dst_hbm, cnt_hbm, idx_v, cnt_v):
    cid = lax.axis_index("c")
    sid = lax.axis_index("s")
    wid = sid * NC + cid

    zvec = jnp.zeros((L,), jnp.float32)
    def zero_cnt(k, _):
      cnt_v[pl.ds(k * L, L)] = zvec
      return 0
    lax.fori_loop(0, n_pad // L, zero_cnt, 0)

    def block(ib, _):
      row0 = wid * chunks_per_tile + ib * CIB
      pltpu.sync_copy(dst_hbm.at[pl.ds(row0, CIB)], idx_v)
      def chunk(j, _):
        for c8 in range(CCHUNK // L):
          d = idx_v[j, pl.ds(c8 * L, L)]
          occ, last = plsc.scan_count(d)
          plsc.addupdate_scatter(
              cnt_v, [d], occ.astype(jnp.float32), mask=last)
        return 0
      lax.fori_loop(0, CIB, chunk, 0)
      return 0
    lax.fori_loop(0, blocks, block, 0)

    pltpu.sync_copy(cnt_v, cnt_hbm.at[cid, sid])

  return sc_counts


def _sc_edges(n_pad2, chunks_per_tile):
  mesh = plsc.VectorSubcoreMesh(core_axis_name="c", subcore_axis_name="s",
                                num_cores=NC, num_subcores=NS)
  rows_per_tile = n_pad2 // NS
  zcopies = rows_per_tile // CHUNK
  blocks_per_tile = chunks_per_tile // IB
  nb2 = blocks_per_tile // 2
  U = 2 * IB

  @functools.partial(
      pl.kernel,
      out_type=jax.ShapeDtypeStruct((NC, n_pad2, SROW), jnp.float32),
      mesh=mesh,
      compiler_params=pltpu.CompilerParams(needs_layout_passes=False,
                                           use_tc_tiling_on_sc=False),
      scratch_types=[
          pltpu.VMEM((IB, CHUNK), jnp.int32),
          pltpu.VMEM((IB, CHUNK), jnp.int32),
          pltpu.VMEM((IB, CHUNK), jnp.int32),
          pltpu.VMEM((IB, CHUNK), jnp.int32),
          pltpu.VMEM((IB, CHUNK), jnp.int32),
          pltpu.VMEM((IB, CHUNK), jnp.int32),
          pltpu.VMEM((IB * CHUNK + L,), jnp.float32),
          pltpu.VMEM((IB * CHUNK + L,), jnp.float32),
          pltpu.VMEM((CHUNK, H2), jnp.float32),
          pltpu.VMEM((CHUNK, H2), jnp.float32),
          pltpu.VMEM((CHUNK, H2), jnp.float32),
          pltpu.VMEM((CHUNK, H2), jnp.float32),
          pltpu.VMEM((CHUNK, SROW), jnp.float32),
          pltpu.VMEM((CHUNK, SROW), jnp.float32),
          pltpu.VMEM_SHARED((n_pad2, SROW), jnp.float32),
          pltpu.SemaphoreType.DMA,
          pltpu.SemaphoreType.DMA,
          pltpu.SemaphoreType.DMA,
          pltpu.SemaphoreType.DMA,
          pltpu.SemaphoreType.DMA,
          pltpu.SemaphoreType.DMA,
          pltpu.SemaphoreType.DMA,
          pltpu.SemaphoreType.DMA,
      ],
  )
  def sc_edges(a_hbm, b_hbm, srcoff_hbm, dstoff_hbm, dstsh_hbm, par_hbm,
               out_hbm,
               src_v0, src_v1, dstg_v0, dstg_v1, dsts_v0, dsts_v1,
               par_v0, par_v1, a_b0, a_b1, b_b0, b_b1, g_b0, g_b1, acc,
               sem_a0, sem_a1, sem_b0, sem_b1, sem_s0, sem_s1,
               sem_i0, sem_i1):
    srcs = (src_v0, src_v1)
    dstgs = (dstg_v0, dstg_v1)
    dstss = (dsts_v0, dsts_v1)
    pars = (par_v0, par_v1)
    abufs = (a_b0, a_b1)
    bbufs = (b_b0, b_b1)
    gbufs = (g_b0, g_b1)
    sas = (sem_a0, sem_a1)
    sbs = (sem_b0, sem_b1)
    sss = (sem_s0, sem_s1)
    sis = (sem_i0, sem_i1)

    cid = lax.axis_index("c")
    sid = lax.axis_index("s")
    base = sid * chunks_per_tile

    def issue_idx(block, slot, sem):
      row0 = base + block * IB
      pltpu.async_copy(srcoff_hbm.at[cid].at[pl.ds(row0, IB)], srcs[slot], sem)
      pltpu.async_copy(dstoff_hbm.at[cid].at[pl.ds(row0, IB)], dstgs[slot], sem)
      pltpu.async_copy(dstsh_hbm.at[pl.ds(row0, IB)], dstss[slot], sem)
      pltpu.async_copy(par_hbm.at[pl.ds(row0 * CHUNK, IB * CHUNK)],
                       pars[slot].at[pl.ds(0, IB * CHUNK)], sem)

    def wait_idx(slot):
      for ref in (srcs[slot], dstgs[slot], dstss[slot]):
        pltpu.make_async_copy(dstsh_hbm.at[pl.ds(base, IB)], ref,
                              sis[slot]).wait()
      pltpu.make_async_copy(par_hbm.at[pl.ds(0, IB * CHUNK)],
                            pars[slot].at[pl.ds(0, IB * CHUNK)],
                            sis[slot]).wait()

    def issue_gather(slot, row, buf_p):
      pltpu.async_copy(a_hbm.at[srcs[slot].at[row]], abufs[buf_p], sas[buf_p])
      pltpu.async_copy(b_hbm.at[dstgs[slot].at[row]], bbufs[buf_p], sbs[buf_p])

    def wait_gather(slot, row, buf_p):
      pltpu.make_async_copy(a_hbm.at[srcs[slot].at[row]], abufs[buf_p],
                            sas[buf_p]).wait()
      pltpu.make_async_copy(b_hbm.at[dstgs[slot].at[row]], bbufs[buf_p],
                            sbs[buf_p]).wait()

    def wait_scatter(buf_p):
      pltpu.make_async_copy(gbufs[buf_p], acc.at[dstss[0].at[0]],
                            sss[buf_p]).wait()

    zvec = jnp.zeros((L,), jnp.float32)
    def zero_row(r, _):
      for c8 in range(SROW // L):
        g_b0[r, pl.ds(c8 * L, L)] = zvec
      return 0
    lax.fori_loop(0, CHUNK, zero_row, 0)
    base_row = sid * rows_per_tile
    for z in range(zcopies):
      pltpu.sync_copy(g_b0, acc.at[pl.ds(base_row + z * CHUNK, CHUNK)])
    plsc.subcore_barrier()

    issue_idx(0, 0, sis[0])
    wait_idx(0)
    issue_idx(1, 1, sis[1])
    issue_gather(0, 0, 0)

    def outer(kb2, _):
      for j in range(U):
        p = j % 2
        slot = j // IB
        row = j % IB
        p1 = 1 - p

        if j == 2:
          @pl.when(kb2 > 0)
          def _():
            issue_idx(2 * kb2 + 1, 1, sis[1])
        if j == 9:
          @pl.when(kb2 < nb2 - 1)
          def _():
            issue_idx(2 * kb2 + 2, 0, sis[0])
        if j == IB - 1:
          wait_idx(1)

        if j == 0:
          @pl.when(kb2 > 0)
          def _():
            wait_scatter(p1)
          issue_gather(0, 1, p1)
        elif j == U - 1:
          @pl.when(kb2 < nb2 - 1)
          def _():
            wait_idx(0)
            wait_scatter(p1)
            issue_gather(0, 0, p1)
        else:
          wait_scatter(p1)
          issue_gather((j + 1) // IB, (j + 1) % IB, p1)

        wait_gather(slot, row, p)

        def gelu_row(r, _):
          pb = pars[slot][pl.ds(row * CHUNK + r, L)][0]
          pe = 1.0 - pb
          for c4 in range(H2 // L):
            x = (abufs[p][r, pl.ds(c4 * L, L)]
                 + bbufs[p][r, pl.ds(c4 * L, L)])
            g = _gelu_fast(x)
            gbufs[p][r, pl.ds(c4 * L, L)] = g * pe
            gbufs[p][r, pl.ds(H2 + c4 * L, L)] = g * pb
          return 0
        lax.fori_loop(0, CHUNK, gelu_row, 0)

        pltpu.async_copy(gbufs[p], acc.at[dstss[slot].at[row]], sss[p],
                         add=True)
      return 0
    lax.fori_loop(0, nb2, outer, 0)

    wait_scatter(0)
    wait_scatter(1)
    plsc.subcore_barrier()

    @pl.when(sid == 0)
    def _():
      pltpu.sync_copy(acc, out_hbm.at[cid])

  return sc_edges


def _post_body(s0_ref, s1_ref, c_ref, h_ref, w2_ref, b2_ref, w3_ref, b3_ref,
               w4_ref, b4_ref, gm_ref, bt_ref, o_ref):
  gsum = jnp.concatenate([s0_ref[0], s1_ref[0]], axis=1)
  counts = jnp.sum(c_ref[...], axis=1)[:, None]
  h = h_ref[...]
  msum = jnp.dot(gsum, w2_ref[...], preferred_element_type=jnp.float32)
  msum = msum + counts * b2_ref[...]
  agg = msum / jnp.maximum(counts, 1.0)
  u = (jnp.dot(h, w3_ref[0:128, :], preferred_element_type=jnp.float32)
       + jnp.dot(agg, w3_ref[128:256, :], preferred_element_type=jnp.float32)
       + b3_ref[...])
  u = _gelu(u)
  hn = jnp.dot(u, w4_ref[...], preferred_element_type=jnp.float32)
  hn = hn + b4_ref[...] + h
  mean = jnp.mean(hn, axis=-1, keepdims=True)
  var = jnp.mean((hn - mean) ** 2, axis=-1, keepdims=True)
  o_ref[...] = ((hn - mean) / jnp.sqrt(var + 1e-5) * gm_ref[...]
                + bt_ref[...])


def _post(s_flat, cnts, h, w2, b2, w3, b3, w4, b4, gamma, beta, n, hdim):
  blk = 1000
  grid = (n // blk,)
  full = lambda shape: pl.BlockSpec(shape, lambda i: tuple(0 for _ in shape))
  return pl.pallas_call(
      _post_body,
      grid=grid,
      in_specs=[
          pl.BlockSpec((1, blk, H2), lambda i: (0, i, 0)),
          pl.BlockSpec((1, blk, H2), lambda i: (1, i, 0)),
          pl.BlockSpec((blk, NW), lambda i: (i, 0)),
          pl.BlockSpec((blk, hdim), lambda i: (i, 0)),
          full((hdim, hdim)),
          full((1, hdim)),
          full((2 * hdim, hdim)),
          full((1, hdim)),
          full((hdim, hdim)),
          full((1, hdim)),
          full((1, hdim)),
          full((1, hdim)),
      ],
      out_specs=pl.BlockSpec((blk, hdim), lambda i: (i, 0)),
      out_shape=jax.ShapeDtypeStruct((n, hdim), jnp.float32),
  )(s_flat, s_flat, cnts, h, w2, b2, w3, b3, w4, b4, gamma, beta)


def kernel(h, edge_index, q_proj, W1, b1, W2, b2, W3, b3, W4, b4, gamma, beta):
  n, hdim = h.shape
  e = edge_index.shape[1]

  u = 2 * IB
  chunks_per_tile = -(-(-(-e // (NS * CHUNK))) // u) * u
  e_per_tile = chunks_per_tile * CHUNK
  e_pad = e_per_tile * NS
  cchunks_per_tile = e_pad // (NW * CCHUNK)
  n_pad = -(-(n + 1) // (NS * CCHUNK)) * (NS * CCHUNK)
  n_pad2 = -(-(n // 2 + 1) // (NS * CHUNK)) * (NS * CHUNK)

  src = edge_index[0]
  dst = edge_index[1]
  pad = e_pad - e
  src_p = jnp.concatenate([src, jnp.zeros((pad,), jnp.int32)])
  dstg_p = jnp.concatenate([dst, jnp.zeros((pad,), jnp.int32)])
  dsts_p = jnp.concatenate([dst, jnp.full((pad,), n, jnp.int32)])

  src2 = src_p.reshape(-1, CHUNK)
  dstg2 = dstg_p.reshape(-1, CHUNK)
  srcoff = jnp.stack([src2, src2 + n])
  dstoff = jnp.stack([dstg2, dstg2 + n])
  dstsh = (dsts_p // 2).reshape(-1, CHUNK)
  par = (dsts_p % 2).astype(jnp.float32)
  dstc = dsts_p.reshape(-1, CCHUNK)

  a, b = _pre(h, W1, q_proj, b1[None, :], n, hdim)
  a2 = jnp.concatenate([a[:, :H2], a[:, H2:]], axis=0)
  b2h = jnp.concatenate([b[:, :H2], b[:, H2:]], axis=0)

  cnt_parts = _sc_counts(n_pad, cchunks_per_tile)(dstc)
  s_parts = _sc_edges(n_pad2, chunks_per_tile)(
      a2, b2h, srcoff, dstoff, dstsh, par)

  cnts = cnt_parts.reshape(NW, n_pad).T
  s_flat = s_parts.reshape(NC, n_pad2 * 2, H2)

  return _post(s_flat, cnts, h, W2, b2[None, :], W3, b3[None, :],
               W4, b4[None, :], gamma[None, :], beta[None, :], n, hdim)


# speedup vs baseline: 2.2616x; 2.2616x over previous
import functools

import jax
import jax.numpy as jnp
from jax import lax
from jax.experimental import pallas as pl
from jax.experimental.pallas import tpu as pltpu
from jax.experimental.pallas import tpu_sc as plsc

NC = 1
NS = 16
L = 16
NW = NC * NS

CHUNK = 64
IB = 8
SROW = 128


def _erf(z):
  az = jnp.abs(z)
  t = 1.0 / (1.0 + 0.3275911 * az)
  poly = t * (0.254829592 + t * (-0.284496736 + t * (1.421413741
              + t * (-1.453152027 + t * 1.061405429))))
  e = poly * jnp.exp(-az * az)
  pos = 1.0 - e
  return jnp.where(z < 0.0, -pos, pos)


def _gelu(x):
  return 0.5 * x * (1.0 + _erf(x * 0.7071067811865476))


def _gelu_fast(x):
  return x / (1.0 + jnp.exp(-1.702 * x))


def _pre_body(h_ref, w1_ref, q_ref, b1_ref, a_ref, b_ref):
  h = h_ref[...]
  w1a = w1_ref[0:128, :]
  w1b = w1_ref[128:256, :]
  w1c = w1_ref[256:384, :]
  qc = jnp.dot(q_ref[...], w1c, preferred_element_type=jnp.float32) + b1_ref[...]
  a_ref[...] = jnp.dot(h, w1a, preferred_element_type=jnp.float32) + qc
  b_ref[...] = jnp.dot(h, w1b, preferred_element_type=jnp.float32)


def _pre(h, w1, q, b1, n, hdim):
  blk = 1000
  grid = (n // blk,)
  return pl.pallas_call(
      _pre_body,
      grid=grid,
      in_specs=[
          pl.BlockSpec((blk, hdim), lambda i: (i, 0)),
          pl.BlockSpec((3 * hdim, hdim), lambda i: (0, 0)),
          pl.BlockSpec((1, hdim), lambda i: (0, 0)),
          pl.BlockSpec((1, hdim), lambda i: (0, 0)),
      ],
      out_specs=[
          pl.BlockSpec((blk, hdim), lambda i: (i, 0)),
          pl.BlockSpec((blk, hdim), lambda i: (i, 0)),
      ],
      out_shape=[
          jax.ShapeDtypeStruct((n, hdim), jnp.float32),
          jax.ShapeDtypeStruct((n, hdim), jnp.float32),
      ],
  )(h, w1, q, b1)


def _sc_edges(n_pad, chunks_per_tile):
  mesh = plsc.VectorSubcoreMesh(core_axis_name="c", subcore_axis_name="s",
                                num_cores=NC, num_subcores=NS)
  rows_per_tile = n_pad // NS
  zcopies = rows_per_tile // (2 * CHUNK)

  blocks_per_tile = chunks_per_tile // IB
  nb2 = blocks_per_tile // 2
  U = 2 * IB

  @functools.partial(
      pl.kernel,
      out_type=jax.ShapeDtypeStruct((NC, n_pad, SROW), jnp.float32),
      mesh=mesh,
      compiler_params=pltpu.CompilerParams(needs_layout_passes=False),
      scratch_types=[
          pltpu.VMEM((IB, 2 * CHUNK), jnp.int32),
          pltpu.VMEM((IB, 2 * CHUNK), jnp.int32),
          pltpu.VMEM((IB, CHUNK), jnp.int32),
          pltpu.VMEM((IB, CHUNK), jnp.int32),
          pltpu.VMEM((2 * CHUNK, 128), jnp.float32),
          pltpu.VMEM((2 * CHUNK, 128), jnp.float32),
          pltpu.VMEM_SHARED((n_pad, SROW), jnp.float32),
          pltpu.SemaphoreType.DMA,
          pltpu.SemaphoreType.DMA,
          pltpu.SemaphoreType.DMA,
          pltpu.SemaphoreType.DMA,
          pltpu.SemaphoreType.DMA,
          pltpu.SemaphoreType.DMA,
          pltpu.SemaphoreType.DMA,
          pltpu.SemaphoreType.DMA,
      ],
  )
  def sc_edges(t_hbm, gidx_hbm, dst_hbm, out_hbm,
               gidx_v0, gidx_v1, dst_v0, dst_v1,
               ab_b0, ab_b1, acc,
               sem_a0, sem_a1, sem_b0, sem_b1, sem_s0, sem_s1,
               sem_i0, sem_i1):
    gidxs = (gidx_v0, gidx_v1)
    dstss = (dst_v0, dst_v1)
    abufs = (ab_b0, ab_b1)
    sas = (sem_a0, sem_a1)
    sss = (sem_s0, sem_s1)
    sis = (sem_i0, sem_i1)

    cid = lax.axis_index("c")
    sid = lax.axis_index("s")
    wid = sid * NC + cid
    base = wid * chunks_per_tile

    def issue_idx(block, slot, sem):
      row0 = base + block * IB
      pltpu.async_copy(gidx_hbm.at[pl.ds(row0, IB)], gidxs[slot], sem)
      pltpu.async_copy(dst_hbm.at[pl.ds(row0, IB)], dstss[slot], sem)

    def wait_idx(slot):
      pltpu.make_async_copy(gidx_hbm.at[pl.ds(base, IB)], gidxs[slot],
                            sis[slot]).wait()
      pltpu.make_async_copy(dst_hbm.at[pl.ds(base, IB)], dstss[slot],
                            sis[slot]).wait()

    def issue_gather(slot, row, buf_p):
      pltpu.async_copy(t_hbm.at[gidxs[slot].at[row]], abufs[buf_p],
                       sas[buf_p])

    def wait_gather(slot, row, buf_p):
      pltpu.make_async_copy(t_hbm.at[gidxs[slot].at[row]], abufs[buf_p],
                            sas[buf_p]).wait()

    def wait_scatter(slot, row, buf_p):
      pltpu.make_async_copy(abufs[buf_p].at[pl.ds(0, CHUNK)],
                            acc.at[dstss[slot].at[row]],
                            sss[buf_p]).wait()

    zvec = jnp.zeros((L,), jnp.float32)
    def zero_row(r, _):
      for c8 in range(SROW // L):
        ab_b0[r, pl.ds(c8 * L, L)] = zvec
      return 0
    lax.fori_loop(0, 2 * CHUNK, zero_row, 0)
    base_row = sid * rows_per_tile
    for z in range(zcopies):
      pltpu.sync_copy(
          ab_b0, acc.at[pl.ds(base_row + z * 2 * CHUNK, 2 * CHUNK)])
    plsc.subcore_barrier()

    issue_idx(0, 0, sis[0])
    wait_idx(0)
    issue_idx(1, 1, sis[1])
    issue_gather(0, 0, 0)

    def outer(kb2, _):
      for j in range(U):
        p = j % 2
        slot = j // IB
        row = j % IB
        p1 = 1 - p

        if j == 2:
          @pl.when(kb2 > 0)
          def _():
            issue_idx(2 * kb2 + 1, 1, sis[1])
        if j == 9:
          @pl.when(kb2 < nb2 - 1)
          def _():
            issue_idx(2 * kb2 + 2, 0, sis[0])
        if j == IB - 1:
          wait_idx(1)

        if j == 0:
          @pl.when(kb2 > 0)
          def _():
            wait_scatter(0, 0, p1)
          issue_gather(0, 1, p1)
        elif j == U - 1:
          @pl.when(kb2 < nb2 - 1)
          def _():
            wait_idx(0)
            wait_scatter(0, 0, p1)
            issue_gather(0, 0, p1)
        else:
          wait_scatter(0, 0, p1)
          issue_gather((j + 1) // IB, (j + 1) % IB, p1)

        wait_gather(slot, row, p)

        def gelu_row(r, _):
          for c8 in range(128 // L):
            x = (abufs[p][r, pl.ds(c8 * L, L)]
                 + abufs[p][CHUNK + r, pl.ds(c8 * L, L)])
            abufs[p][r, pl.ds(c8 * L, L)] = _gelu_fast(x)
          return 0
        lax.fori_loop(0, CHUNK, gelu_row, 0)

        pltpu.async_copy(abufs[p].at[pl.ds(0, CHUNK)],
                         acc.at[dstss[slot].at[row]], sss[p], add=True)
      return 0
    lax.fori_loop(0, nb2, outer, 0)

    wait_scatter(0, 0, 0)
    wait_scatter(0, 0, 1)
    plsc.subcore_barrier()

    @pl.when(sid == 0)
    def _():
      pltpu.sync_copy(acc, out_hbm.at[cid])

  return sc_edges


CCHUNK = 128
CIB = 8


def _sc_counts(n_pad, chunks_per_tile):
  mesh = plsc.VectorSubcoreMesh(core_axis_name="c", subcore_axis_name="s",
                                num_cores=NC, num_subcores=NS)
  blocks = chunks_per_tile // CIB

  @functools.partial(
      pl.kernel,
      out_type=jax.ShapeDtypeStruct((NC, NS, n_pad), jnp.float32),
      mesh=mesh,
      compiler_params=pltpu.CompilerParams(needs_layout_passes=False),
      scratch_types=[
          pltpu.VMEM((CIB, CCHUNK), jnp.int32),
          pltpu.VMEM((n_pad,), jnp.float32),
      ],
  )
  def sc_counts(dst_hbm, cnt_hbm, idx_v, cnt_v):
    cid = lax.axis_index("c")
    sid = lax.axis_index("s")
    wid = sid * NC + cid

    zvec = jnp.zeros((L,), jnp.float32)
    def zero_cnt(k, _):
      cnt_v[pl.ds(k * L, L)] = zvec
      return 0
    lax.fori_loop(0, n_pad // L, zero_cnt, 0)

    def block(ib, _):
      row0 = wid * chunks_per_tile + ib * CIB
      pltpu.sync_copy(dst_hbm.at[pl.ds(row0, CIB)], idx_v)
      def chunk(j, _):
        for c8 in range(CCHUNK // L):
          d = idx_v[j, pl.ds(c8 * L, L)]
          occ, last = plsc.scan_count(d)
          plsc.addupdate_scatter(
              cnt_v, [d], occ.astype(jnp.float32), mask=last)
        return 0
      lax.fori_loop(0, CIB, chunk, 0)
      return 0
    lax.fori_loop(0, blocks, block, 0)

    pltpu.sync_copy(cnt_v, cnt_hbm.at[cid, sid])

  return sc_counts


def _post_body(*refs):
  s_refs = refs[:NC]
  (c_ref, h_ref, w2_ref, b2_ref, w3_ref, b3_ref,
   w4_ref, b4_ref, gm_ref, bt_ref, o_ref) = refs[NC:]
  gsum = s_refs[0][0]
  for p in range(1, NC):
    gsum = gsum + s_refs[p][0]
  counts = jnp.sum(c_ref[...], axis=1)[:, None]
  h = h_ref[...]
  msum = jnp.dot(gsum, w2_ref[...], preferred_element_type=jnp.float32)
  msum = msum + counts * b2_ref[...]
  agg = msum / jnp.maximum(counts, 1.0)
  u = (jnp.dot(h, w3_ref[0:128, :], preferred_element_type=jnp.float32)
       + jnp.dot(agg, w3_ref[128:256, :], preferred_element_type=jnp.float32)
       + b3_ref[...])
  u = _gelu(u)
  hn = jnp.dot(u, w4_ref[...], preferred_element_type=jnp.float32)
  hn = hn + b4_ref[...] + h
  mean = jnp.mean(hn, axis=-1, keepdims=True)
  var = jnp.mean((hn - mean) ** 2, axis=-1, keepdims=True)
  o_ref[...] = ((hn - mean) / jnp.sqrt(var + 1e-5) * gm_ref[...]
                + bt_ref[...])


def _post(s_parts, cnts, h, w2, b2, w3, b3, w4, b4, gamma, beta, n, hdim):
  blk = 1000
  grid = (n // blk,)
  full = lambda shape: pl.BlockSpec(shape, lambda i: tuple(0 for _ in shape))
  return pl.pallas_call(
      _post_body,
      grid=grid,
      in_specs=[
          *[pl.BlockSpec((1, blk, SROW),
                         functools.partial(lambda p, i: (p, i, 0), p))
            for p in range(NC)],
          pl.BlockSpec((blk, NW), lambda i: (i, 0)),
          pl.BlockSpec((blk, hdim), lambda i: (i, 0)),
          full((hdim, hdim)),
          full((1, hdim)),
          full((2 * hdim, hdim)),
          full((1, hdim)),
          full((hdim, hdim)),
          full((1, hdim)),
          full((1, hdim)),
          full((1, hdim)),
      ],
      out_specs=pl.BlockSpec((blk, hdim), lambda i: (i, 0)),
      out_shape=jax.ShapeDtypeStruct((n, hdim), jnp.float32),
  )(*([s_parts] * NC), cnts, h, w2, b2, w3, b3, w4, b4, gamma, beta)


def kernel(h, edge_index, q_proj, W1, b1, W2, b2, W3, b3, W4, b4, gamma, beta):
  n, hdim = h.shape
  e = edge_index.shape[1]

  u = 2 * IB
  chunks_per_tile = -(-(-(-e // (NW * CHUNK))) // u) * u
  e_per_tile = chunks_per_tile * CHUNK
  e_pad = e_per_tile * NW
  n_pad = -(-(n + 1) // (NS * CHUNK)) * (NS * CHUNK)

  src = edge_index[0]
  dst = edge_index[1]
  pad = e_pad - e
  src_p = jnp.concatenate([src, jnp.zeros((pad,), jnp.int32)])
  dst_p = jnp.concatenate([dst, jnp.full((pad,), n, jnp.int32)])
  src2 = src_p.reshape(NW * chunks_per_tile, CHUNK)
  dst2 = dst_p.reshape(NW * chunks_per_tile, CHUNK)
  dstc = dst_p.reshape(-1, 128)
  cchunks_per_tile = dstc.shape[0] // NW

  a, b = _pre(h, W1, q_proj, b1[None, :], n, hdim)
  zrows = jnp.zeros((n_pad - n, hdim), jnp.float32)
  t_pad = jnp.concatenate([a, zrows, b, zrows], axis=0)
  gidx = jnp.concatenate([src2, dst2 + n_pad], axis=1)

  cnt_parts = _sc_counts(n_pad, cchunks_per_tile)(dstc)
  s_parts = _sc_edges(n_pad, chunks_per_tile)(t_pad, gidx, dst2)
  cnts = cnt_parts.reshape(NW, n_pad).T

  return _post(s_parts, cnts, h, W2, b2[None, :], W3, b3[None, :],
               W4, b4[None, :], gamma[None, :], beta[None, :], n, hdim)


# speedup vs baseline: 2.7375x; 1.2104x over previous
import functools

import jax
import jax.numpy as jnp
from jax import lax
from jax.experimental import pallas as pl
from jax.experimental.pallas import tpu as pltpu
from jax.experimental.pallas import tpu_sc as plsc

NC = 1
NS = 16
L = 16
NW = NC * NS

CHUNK = 80
IB = 8
SROW = 128


def _erf(z):
  az = jnp.abs(z)
  t = 1.0 / (1.0 + 0.3275911 * az)
  poly = t * (0.254829592 + t * (-0.284496736 + t * (1.421413741
              + t * (-1.453152027 + t * 1.061405429))))
  e = poly * jnp.exp(-az * az)
  pos = 1.0 - e
  return jnp.where(z < 0.0, -pos, pos)


def _gelu(x):
  return 0.5 * x * (1.0 + _erf(x * 0.7071067811865476))


def _gelu_fast(x):
  return x / (1.0 + jnp.exp(-1.702 * x))


def _pre_body(h_ref, w1_ref, q_ref, b1_ref, a_ref, b_ref):
  h = h_ref[...]
  w1a = w1_ref[0:128, :]
  w1b = w1_ref[128:256, :]
  w1c = w1_ref[256:384, :]
  qc = jnp.dot(q_ref[...], w1c, preferred_element_type=jnp.float32) + b1_ref[...]
  a_ref[...] = jnp.dot(h, w1a, preferred_element_type=jnp.float32) + qc
  b_ref[...] = jnp.dot(h, w1b, preferred_element_type=jnp.float32)


def _pre(h, w1, q, b1, n, hdim):
  blk = 1000
  grid = (n // blk,)
  return pl.pallas_call(
      _pre_body,
      grid=grid,
      in_specs=[
          pl.BlockSpec((blk, hdim), lambda i: (i, 0)),
          pl.BlockSpec((3 * hdim, hdim), lambda i: (0, 0)),
          pl.BlockSpec((1, hdim), lambda i: (0, 0)),
          pl.BlockSpec((1, hdim), lambda i: (0, 0)),
      ],
      out_specs=[
          pl.BlockSpec((blk, hdim), lambda i: (i, 0)),
          pl.BlockSpec((blk, hdim), lambda i: (i, 0)),
      ],
      out_shape=[
          jax.ShapeDtypeStruct((n, hdim), jnp.float32),
          jax.ShapeDtypeStruct((n, hdim), jnp.float32),
      ],
  )(h, w1, q, b1)


def _sc_edges(n_pad, chunks_per_tile):
  mesh = plsc.VectorSubcoreMesh(core_axis_name="c", subcore_axis_name="s",
                                num_cores=NC, num_subcores=NS)
  rows_per_tile = n_pad // NS
  ZROWS = 80
  zcopies = rows_per_tile // ZROWS

  blocks_per_tile = chunks_per_tile // IB
  nb2 = blocks_per_tile // 2
  U = 2 * IB

  @functools.partial(
      pl.kernel,
      out_type=jax.ShapeDtypeStruct((NC, n_pad, SROW), jnp.float32),
      mesh=mesh,
      compiler_params=pltpu.CompilerParams(needs_layout_passes=False),
      scratch_types=[
          pltpu.VMEM((IB, CHUNK), jnp.int32),
          pltpu.VMEM((IB, CHUNK), jnp.int32),
          pltpu.VMEM((IB, CHUNK), jnp.int32),
          pltpu.VMEM((IB, CHUNK), jnp.int32),
          pltpu.VMEM((CHUNK, 128), jnp.float32),
          pltpu.VMEM((CHUNK, 128), jnp.float32),
          pltpu.VMEM((CHUNK, 128), jnp.float32),
          pltpu.VMEM((CHUNK, 128), jnp.float32),
          pltpu.VMEM_SHARED((n_pad, SROW), jnp.float32),
          pltpu.SemaphoreType.DMA,
          pltpu.SemaphoreType.DMA,
          pltpu.SemaphoreType.DMA,
          pltpu.SemaphoreType.DMA,
          pltpu.SemaphoreType.DMA,
          pltpu.SemaphoreType.DMA,
          pltpu.SemaphoreType.DMA,
          pltpu.SemaphoreType.DMA,
      ],
  )
  def sc_edges(a_hbm, b_hbm, src_hbm, dst_hbm, out_hbm,
               src_v0, src_v1, dst_v0, dst_v1,
               a_b0, a_b1, b_b0, b_b1, acc,
               sem_a0, sem_a1, sem_b0, sem_b1, sem_s0, sem_s1,
               sem_i0, sem_i1):
    srcs = (src_v0, src_v1)
    dstgs = (dst_v0, dst_v1)
    dstss = (dst_v0, dst_v1)
    abufs = (a_b0, a_b1)
    bbufs = (b_b0, b_b1)
    sas = (sem_a0, sem_a1)
    sbs = (sem_b0, sem_b1)
    sss = (sem_s0, sem_s1)
    sis = (sem_i0, sem_i1)

    cid = lax.axis_index("c")
    sid = lax.axis_index("s")
    wid = sid * NC + cid
    base = wid * chunks_per_tile

    def issue_idx(block, slot, sem):
      row0 = base + block * IB
      pltpu.async_copy(src_hbm.at[pl.ds(row0, IB)], srcs[slot], sem)
      pltpu.async_copy(dst_hbm.at[pl.ds(row0, IB)], dstgs[slot], sem)

    def wait_idx(slot):
      for ref in (srcs[slot], dstgs[slot]):
        pltpu.make_async_copy(src_hbm.at[pl.ds(base, IB)], ref,
                              sis[slot]).wait()

    def issue_gather(slot, row, buf_p):
      pltpu.async_copy(a_hbm.at[srcs[slot].at[row]], abufs[buf_p],
                       sas[buf_p])
      pltpu.async_copy(b_hbm.at[dstgs[slot].at[row]], bbufs[buf_p],
                       sbs[buf_p])

    def wait_gather(slot, row, buf_p):
      pltpu.make_async_copy(a_hbm.at[srcs[slot].at[row]], abufs[buf_p],
                            sas[buf_p]).wait()
      pltpu.make_async_copy(b_hbm.at[dstgs[slot].at[row]], bbufs[buf_p],
                            sbs[buf_p]).wait()

    def wait_scatter(slot, row, buf_p):
      pltpu.make_async_copy(abufs[buf_p], acc.at[dstss[slot].at[row]],
                            sss[buf_p]).wait()

    zvec = jnp.zeros((L,), jnp.float32)
    def zero_row(r, _):
      for c8 in range(SROW // L):
        a_b0[r, pl.ds(c8 * L, L)] = zvec
      return 0
    lax.fori_loop(0, ZROWS, zero_row, 0)
    base_row = sid * rows_per_tile
    for z in range(zcopies):
      pltpu.sync_copy(a_b0.at[pl.ds(0, ZROWS)],
                      acc.at[pl.ds(base_row + z * ZROWS, ZROWS)])
    plsc.subcore_barrier()

    issue_idx(0, 0, sis[0])
    wait_idx(0)
    issue_idx(1, 1, sis[1])
    issue_gather(0, 0, 0)

    def outer(kb2, _):
      for j in range(U):
        p = j % 2
        slot = j // IB
        row = j % IB
        p1 = 1 - p

        if j == 2:
          @pl.when(kb2 > 0)
          def _():
            issue_idx(2 * kb2 + 1, 1, sis[1])
        if j == 9:
          @pl.when(kb2 < nb2 - 1)
          def _():
            issue_idx(2 * kb2 + 2, 0, sis[0])
        if j == IB - 1:
          wait_idx(1)

        if j == 0:
          @pl.when(kb2 > 0)
          def _():
            wait_scatter(0, 0, p1)
          issue_gather(0, 1, p1)
        elif j == U - 1:
          @pl.when(kb2 < nb2 - 1)
          def _():
            wait_idx(0)
            wait_scatter(0, 0, p1)
            issue_gather(0, 0, p1)
        else:
          wait_scatter(0, 0, p1)
          issue_gather((j + 1) // IB, (j + 1) % IB, p1)

        wait_gather(slot, row, p)

        def gelu_row(r2, _):
          for dr in range(2):
            r = 2 * r2 + dr
            for c8 in range(128 // L):
              x = (abufs[p][r, pl.ds(c8 * L, L)]
                   + bbufs[p][r, pl.ds(c8 * L, L)])
              abufs[p][r, pl.ds(c8 * L, L)] = _gelu_fast(x)
          return 0
        lax.fori_loop(0, CHUNK // 2, gelu_row, 0)

        pltpu.async_copy(abufs[p], acc.at[dstss[slot].at[row]], sss[p],
                         add=True)
      return 0
    lax.fori_loop(0, nb2, outer, 0)

    wait_scatter(0, 0, 0)
    wait_scatter(0, 0, 1)
    plsc.subcore_barrier()

    @pl.when(sid == 0)
    def _():
      pltpu.sync_copy(acc, out_hbm.at[cid])

  return sc_edges


CCHUNK = 128
CIB = 8


def _sc_counts(n_pad, chunks_per_tile):
  mesh = plsc.VectorSubcoreMesh(core_axis_name="c", subcore_axis_name="s",
                                num_cores=NC, num_subcores=NS)
  blocks = chunks_per_tile // CIB

  @functools.partial(
      pl.kernel,
      out_type=jax.ShapeDtypeStruct((NC, NS, n_pad), jnp.float32),
      mesh=mesh,
      compiler_params=pltpu.CompilerParams(needs_layout_passes=False),
      scratch_types=[
          pltpu.VMEM((CIB, CCHUNK), jnp.int32),
          pltpu.VMEM((n_pad,), jnp.float32),
      ],
  )
  def sc_counts(dst_hbm, cnt_hbm, idx_v, cnt_v):
    cid = lax.axis_index("c")
    sid = lax.axis_index("s")
    wid = sid * NC + cid

    zvec = jnp.zeros((L,), jnp.float32)
    def zero_cnt(k, _):
      cnt_v[pl.ds(k * L, L)] = zvec
      return 0
    lax.fori_loop(0, n_pad // L, zero_cnt, 0)

    def block(ib, _):
      row0 = wid * chunks_per_tile + ib * CIB
      pltpu.sync_copy(dst_hbm.at[pl.ds(row0, CIB)], idx_v)
      def chunk(j, _):
        for c8 in range(CCHUNK // L):
          d = idx_v[j, pl.ds(c8 * L, L)]
          occ, last = plsc.scan_count(d)
          plsc.addupdate_scatter(
              cnt_v, [d], occ.astype(jnp.float32), mask=last)
        return 0
      lax.fori_loop(0, CIB, chunk, 0)
      return 0
    lax.fori_loop(0, blocks, block, 0)

    pltpu.sync_copy(cnt_v, cnt_hbm.at[cid, sid])

  return sc_counts


def _post_body(*refs):
  s_refs = refs[:NC]
  (c_ref, h_ref, w2_ref, b2_ref, w3_ref, b3_ref,
   w4_ref, b4_ref, gm_ref, bt_ref, o_ref) = refs[NC:]
  gsum = s_refs[0][0]
  for p in range(1, NC):
    gsum = gsum + s_refs[p][0]
  counts = jnp.sum(c_ref[...], axis=1)[:, None]
  h = h_ref[...]
  msum = jnp.dot(gsum, w2_ref[...], preferred_element_type=jnp.float32)
  msum = msum + counts * b2_ref[...]
  agg = msum / jnp.maximum(counts, 1.0)
  u = (jnp.dot(h, w3_ref[0:128, :], preferred_element_type=jnp.float32)
       + jnp.dot(agg, w3_ref[128:256, :], preferred_element_type=jnp.float32)
       + b3_ref[...])
  u = _gelu(u)
  hn = jnp.dot(u, w4_ref[...], preferred_element_type=jnp.float32)
  hn = hn + b4_ref[...] + h
  mean = jnp.mean(hn, axis=-1, keepdims=True)
  var = jnp.mean((hn - mean) ** 2, axis=-1, keepdims=True)
  o_ref[...] = ((hn - mean) / jnp.sqrt(var + 1e-5) * gm_ref[...]
                + bt_ref[...])


def _post(s_parts, cnts, h, w2, b2, w3, b3, w4, b4, gamma, beta, n, hdim):
  blk = 1000
  grid = (n // blk,)
  full = lambda shape: pl.BlockSpec(shape, lambda i: tuple(0 for _ in shape))
  return pl.pallas_call(
      _post_body,
      grid=grid,
      in_specs=[
          *[pl.BlockSpec((1, blk, SROW),
                         functools.partial(lambda p, i: (p, i, 0), p))
            for p in range(NC)],
          pl.BlockSpec((blk, NW), lambda i: (i, 0)),
          pl.BlockSpec((blk, hdim), lambda i: (i, 0)),
          full((hdim, hdim)),
          full((1, hdim)),
          full((2 * hdim, hdim)),
          full((1, hdim)),
          full((hdim, hdim)),
          full((1, hdim)),
          full((1, hdim)),
          full((1, hdim)),
      ],
      out_specs=pl.BlockSpec((blk, hdim), lambda i: (i, 0)),
      out_shape=jax.ShapeDtypeStruct((n, hdim), jnp.float32),
  )(*([s_parts] * NC), cnts, h, w2, b2, w3, b3, w4, b4, gamma, beta)


def kernel(h, edge_index, q_proj, W1, b1, W2, b2, W3, b3, W4, b4, gamma, beta):
  n, hdim = h.shape
  e = edge_index.shape[1]

  u = 2 * IB
  chunks_per_tile = -(-(-(-e // (NW * CHUNK))) // u) * u
  e_per_tile = chunks_per_tile * CHUNK
  e_pad = e_per_tile * NW
  n_pad = -(-(n + 1) // (NS * CHUNK)) * (NS * CHUNK)

  src = edge_index[0]
  dst = edge_index[1]
  pad = e_pad - e
  src_p = jnp.concatenate([src, jnp.zeros((pad,), jnp.int32)])
  dst_p = jnp.concatenate([dst, jnp.full((pad,), n, jnp.int32)])
  src2 = src_p.reshape(NW * chunks_per_tile, CHUNK)
  dst2 = dst_p.reshape(NW * chunks_per_tile, CHUNK)
  crows = -(-e_pad // 128)
  crows = -(-crows // (NW * 8)) * (NW * 8)
  dstc = jnp.concatenate(
      [dst_p, jnp.full((crows * 128 - e_pad,), n, jnp.int32)]).reshape(-1, 128)
  cchunks_per_tile = dstc.shape[0] // NW

  a, b = _pre(h, W1, q_proj, b1[None, :], n, hdim)
  b_pad = jnp.concatenate(
      [b, jnp.zeros((n_pad - n, hdim), jnp.float32)], axis=0)

  cnt_parts = _sc_counts(n_pad, cchunks_per_tile)(dstc)
  s_parts = _sc_edges(n_pad, chunks_per_tile)(a, b_pad, src2, dst2)
  cnts = cnt_parts.reshape(NW, n_pad).T

  return _post(s_parts, cnts, h, W2, b2[None, :], W3, b3[None, :],
               W4, b4[None, :], gamma[None, :], beta[None, :], n, hdim)
